# baseline (speedup 1.0000x reference)
import functools

import jax
import jax.numpy as jnp
from jax import lax
from jax.experimental import pallas as pl
from jax.experimental.pallas import tpu as pltpu
from jax.experimental.pallas import tpu_sc as plsc

_VOCAB = 100000
_HIDDEN = 128
_SEQ = 200
_BATCH = 1024
_ROWS = _BATCH * _SEQ
_NW = 32
_SEQ_PER_W = _BATCH // _NW
_ROWS_PER_W = _ROWS // _NW
_CHUNK = 128
_POS_PER_CHUNK = 8
_SEQ_PER_CHUNK = _CHUNK // _POS_PER_CHUNK
_NCHUNK = _ROWS_PER_W // _CHUNK


def _lane_sum(v):
    i = lax.iota(jnp.int32, 16)
    dnums = lax.GatherDimensionNumbers(
        offset_dims=(), collapsed_slice_dims=(0,), start_index_map=(0,))
    for d in (1, 2, 4, 8):
        perm = lax.gather(
            v, (i ^ d)[:, None], dnums, (1,),
            mode=lax.GatherScatterMode.PROMISE_IN_BOUNDS)
        v = v + perm
    return v


def _rsqrt(d):
    i = lax.bitcast_convert_type(d, jnp.int32)
    i = jnp.int32(0x5F3759DF) - (i >> 1)
    y = lax.bitcast_convert_type(i, jnp.float32)
    for _ in range(1):
        y = y * (1.5 - 0.5 * d * y * y)
    return y


def _sc_body(table_hbm, ids_hbm, pos_hbm, oidx_hbm, out_hbm, ids_v, pos_v,
             oidx_v, rows0, rows1, outv0, outv1, gsem0, gsem1, osem0, osem1):
    wid = lax.axis_index("s") * 2 + lax.axis_index("c")
    row0 = wid * _ROWS_PER_W
    rows = (rows0, rows1)
    outs = (outv0, outv1)
    gsems = (gsem0, gsem1)
    osems = (osem0, osem1)

    pltpu.sync_copy(ids_hbm.at[pl.ds(row0, _ROWS_PER_W)], ids_v)
    pltpu.sync_copy(oidx_hbm.at[pl.ds(wid * _NCHUNK, _NCHUNK)], oidx_v)
    pltpu.sync_copy(pos_hbm.at[pl.ds(0, _SEQ * _HIDDEN)], pos_v)

    pltpu.async_copy(table_hbm.at[ids_v.at[pl.ds(0, _CHUNK)]], rows0, gsem0)

    def group_body(i, _):
        for b in (0, 1):
            c = 2 * i + b
            pltpu.make_async_copy(
                table_hbm.at[ids_v.at[pl.ds(c * _CHUNK, _CHUNK)]],
                rows[b], gsems[b]).wait()

            @pl.when(c + 1 < _NCHUNK)
            def _():
                idx = ids_v.at[pl.ds((c + 1) * _CHUNK, _CHUNK)]
                pltpu.async_copy(table_hbm.at[idx], rows[b ^ 1], gsems[b ^ 1])

            @pl.when(i >= 1)
            def _():
                pltpu.make_async_copy(
                    outs[b], out_hbm.at[oidx_v.at[c - 2, 0]], osems[b]).wait()

            rows_v = rows[b]
            out_v = outs[b]

            for j in range(_POS_PER_CHUNK):
                p = (c % 25) * _POS_PER_CHUNK + j
                pb = p * _HIDDEN
                ps = [pos_v[pl.ds(pb + 16 * k, 16)]
                      for k in range(_HIDDEN // 16)]

                @plsc.parallel_loop(0, _SEQ_PER_CHUNK, unroll=2)
                def row_body(s):
                    r = j * _SEQ_PER_CHUNK + s
                    xs = []
                    acc = None
                    sq = None
                    for k in range(_HIDDEN // 16):
                        v = rows_v[r, pl.ds(16 * k, 16)] + ps[k]
                        xs.append(v)
                        acc = v if acc is None else acc + v
                        sq = v * v if sq is None else sq + v * v
                    tot = _lane_sum(acc)
                    totsq = _lane_sum(sq)
                    mean = tot * (1.0 / _HIDDEN)
                    var = totsq * (1.0 / _HIDDEN) - mean * mean
                    rinv = _rsqrt(var + 1e-3)
                    for k in range(_HIDDEN // 16):
                        out_v[s, pl.ds(j * _HIDDEN + 16 * k, 16)] = (
                            (xs[k] - mean) * rinv)

            pltpu.async_copy(out_v, out_hbm.at[oidx_v.at[c, 0]], osems[b])
        return 0

    lax.fori_loop(0, _NCHUNK // 2, group_body, 0)

    for b, cl in ((0, _NCHUNK - 2), (1, _NCHUNK - 1)):
        pltpu.make_async_copy(
            outs[b], out_hbm.at[oidx_v.at[cl, 0]], osems[b]).wait()


@jax.jit
def _sc_call(word_embeddings, ids_t, pos_flat, oidx):
    mesh = plsc.VectorSubcoreMesh(core_axis_name="c", subcore_axis_name="s")
    fn = functools.partial(
        pl.kernel,
        mesh=mesh,
        out_type=jax.ShapeDtypeStruct(
            (_ROWS // _POS_PER_CHUNK, _POS_PER_CHUNK * _HIDDEN), jnp.float32),
        scratch_types=[
            pltpu.VMEM((_ROWS_PER_W,), jnp.int32),
            pltpu.VMEM((_SEQ * _HIDDEN,), jnp.float32),
            pltpu.VMEM((_NCHUNK, 1, _SEQ_PER_CHUNK), jnp.int32),
            pltpu.VMEM((_CHUNK, _HIDDEN), jnp.float32),
            pltpu.VMEM((_CHUNK, _HIDDEN), jnp.float32),
            pltpu.VMEM((_SEQ_PER_CHUNK, _POS_PER_CHUNK * _HIDDEN), jnp.float32),
            pltpu.VMEM((_SEQ_PER_CHUNK, _POS_PER_CHUNK * _HIDDEN), jnp.float32),
            pltpu.SemaphoreType.DMA,
            pltpu.SemaphoreType.DMA,
            pltpu.SemaphoreType.DMA,
            pltpu.SemaphoreType.DMA,
        ],
    )(_sc_body)
    return fn(word_embeddings, ids_t, pos_flat, oidx)


def kernel(input_ids, token_type_ids, word_embeddings, position_embeddings):
    del token_type_ids
    b, s = input_ids.shape
    ids_t = (input_ids.astype(jnp.int32)
             .reshape(_NW, 2, _SEQ_PER_CHUNK, _SEQ // _POS_PER_CHUNK,
                      _POS_PER_CHUNK)
             .transpose(0, 1, 3, 4, 2)
             .reshape(-1))
    npb = _SEQ // _POS_PER_CHUNK
    w_ix = jnp.arange(_NW, dtype=jnp.int32)[:, None, None, None]
    h_ix = jnp.arange(2, dtype=jnp.int32)[None, :, None, None]
    pb_ix = jnp.arange(npb, dtype=jnp.int32)[None, None, :, None]
    s_ix = jnp.arange(_SEQ_PER_CHUNK, dtype=jnp.int32)[None, None, None, :]
    oidx = ((w_ix * _SEQ_PER_W + h_ix * _SEQ_PER_CHUNK + s_ix) * npb
            + pb_ix).reshape(_NW * _NCHUNK, 1, _SEQ_PER_CHUNK)
    pos_flat = position_embeddings.reshape(-1)
    out = _sc_call(word_embeddings, ids_t, pos_flat, oidx)
    return out.reshape(b, s, _HIDDEN)


# speedup vs baseline: 2.1662x; 2.1662x over previous
import functools

import jax
import jax.numpy as jnp
from jax import lax
from jax.experimental import pallas as pl
from jax.experimental.pallas import tpu as pltpu
from jax.experimental.pallas import tpu_sc as plsc

_VOCAB = 100000
_HIDDEN = 128
_SEQ = 200
_BATCH = 1024
_ROWS = _BATCH * _SEQ
_NW = 32
_ROWS_PER_W = _ROWS // _NW
_CHUNK = 128
_NCHUNK = _ROWS_PER_W // _CHUNK


def _lane_sum(v):
    i = lax.iota(jnp.int32, 16)
    dnums = lax.GatherDimensionNumbers(
        offset_dims=(), collapsed_slice_dims=(0,), start_index_map=(0,))
    for d in (1, 2, 4, 8):
        perm = lax.gather(
            v, (i ^ d)[:, None], dnums, (1,),
            mode=lax.GatherScatterMode.PROMISE_IN_BOUNDS)
        v = v + perm
    return v


def _tree_sum(vs):
    vs = list(vs)
    while len(vs) > 1:
        vs = [vs[i] + vs[i + 1] for i in range(0, len(vs) - 1, 2)] + (
            [vs[-1]] if len(vs) % 2 else [])
    return vs[0]


def _rsqrt(d):
    i = lax.bitcast_convert_type(d, jnp.int32)
    i = jnp.int32(0x5F3759DF) - (i >> 1)
    y = lax.bitcast_convert_type(i, jnp.float32)
    return y * (1.5 - 0.5 * d * y * y)


def _sc_body(table_hbm, ids_hbm, pos_hbm, out_hbm, ids_v, pos_v,
             rows0, rows1, outv0, outv1, gsem0, gsem1, osem0, osem1):
    wid = lax.axis_index("s") * 2 + lax.axis_index("c")
    row0 = wid * _ROWS_PER_W
    rows = (rows0, rows1)
    outs = (outv0, outv1)
    gsems = (gsem0, gsem1)
    osems = (osem0, osem1)

    pltpu.sync_copy(ids_hbm.at[pl.ds(row0, _ROWS_PER_W)], ids_v)
    pltpu.sync_copy(pos_hbm.at[pl.ds(0, _SEQ * _HIDDEN)], pos_v)

    pltpu.async_copy(table_hbm.at[ids_v.at[pl.ds(0, _CHUNK)]], rows0, gsem0)

    def group_body(i, _):
        for b in (0, 1):
            c = 2 * i + b
            base = row0 + c * _CHUNK
            pltpu.make_async_copy(
                table_hbm.at[ids_v.at[pl.ds(c * _CHUNK, _CHUNK)]],
                rows[b], gsems[b]).wait()

            @pl.when(c + 1 < _NCHUNK)
            def _():
                idx = ids_v.at[pl.ds((c + 1) * _CHUNK, _CHUNK)]
                pltpu.async_copy(table_hbm.at[idx], rows[b ^ 1], gsems[b ^ 1])

            @pl.when(i >= 1)
            def _():
                pltpu.make_async_copy(
                    outs[b],
                    out_hbm.at[pl.ds(base - 2 * _CHUNK, _CHUNK)],
                    osems[b]).wait()

            rows_v = rows[b]
            out_v = outs[b]

            @plsc.parallel_loop(0, _CHUNK, unroll=4)
            def row_body(r):
                p = (base + r) % _SEQ
                pb = p * _HIDDEN
                xs = []
                sqs = []
                for k in range(_HIDDEN // 16):
                    v = (rows_v[r, pl.ds(16 * k, 16)]
                         + pos_v[pl.ds(pb + 16 * k, 16)])
                    xs.append(v)
                    sqs.append(v * v)
                tot = _lane_sum(_tree_sum(xs))
                totsq = _lane_sum(_tree_sum(sqs))
                mean = tot * (1.0 / _HIDDEN)
                var = totsq * (1.0 / _HIDDEN) - mean * mean
                rinv = _rsqrt(var + 1e-3)
                mr = mean * rinv
                for k in range(_HIDDEN // 16):
                    out_v[r, pl.ds(16 * k, 16)] = xs[k] * rinv - mr

            pltpu.async_copy(out_v, out_hbm.at[pl.ds(base, _CHUNK)], osems[b])
        return 0

    lax.fori_loop(0, _NCHUNK // 2, group_body, 0)

    for b, cl in ((0, _NCHUNK - 2), (1, _NCHUNK - 1)):
        pltpu.make_async_copy(
            outs[b], out_hbm.at[pl.ds(row0 + cl * _CHUNK, _CHUNK)],
            osems[b]).wait()


@jax.jit
def _sc_call(word_embeddings, ids_flat, pos_flat):
    mesh = plsc.VectorSubcoreMesh(core_axis_name="c", subcore_axis_name="s")
    fn = functools.partial(
        pl.kernel,
        mesh=mesh,
        out_type=jax.ShapeDtypeStruct((_ROWS, _HIDDEN), jnp.float32),
        scratch_types=[
            pltpu.VMEM((_ROWS_PER_W,), jnp.int32),
            pltpu.VMEM((_SEQ * _HIDDEN,), jnp.float32),
            pltpu.VMEM((_CHUNK, _HIDDEN), jnp.float32),
            pltpu.VMEM((_CHUNK, _HIDDEN), jnp.float32),
            pltpu.VMEM((_CHUNK, _HIDDEN), jnp.float32),
            pltpu.VMEM((_CHUNK, _HIDDEN), jnp.float32),
            pltpu.SemaphoreType.DMA,
            pltpu.SemaphoreType.DMA,
            pltpu.SemaphoreType.DMA,
            pltpu.SemaphoreType.DMA,
        ],
    )(_sc_body)
    return fn(word_embeddings, ids_flat, pos_flat)


def kernel(input_ids, token_type_ids, word_embeddings, position_embeddings):
    del token_type_ids
    b, s = input_ids.shape
    ids_flat = input_ids.reshape(-1).astype(jnp.int32)
    pos_flat = position_embeddings.reshape(-1)
    out = _sc_call(word_embeddings, ids_flat, pos_flat)
    return out.reshape(b, s, _HIDDEN)


# speedup vs baseline: 2.5589x; 1.1813x over previous
import functools

import jax
import jax.numpy as jnp
from jax import lax
from jax.experimental import pallas as pl
from jax.experimental.pallas import tpu as pltpu
from jax.experimental.pallas import tpu_sc as plsc

_VOCAB = 100000
_HIDDEN = 128
_SEQ = 200
_BATCH = 1024
_ROWS = _BATCH * _SEQ
_NW = 32
_ROWS_PER_W = _ROWS // _NW
_CHUNK = 128
_NCHUNK = _ROWS_PER_W // _CHUNK


def _lane_sum(v):
    i = lax.iota(jnp.int32, 16)
    dnums = lax.GatherDimensionNumbers(
        offset_dims=(), collapsed_slice_dims=(0,), start_index_map=(0,))
    for d in (1, 2, 4, 8):
        perm = lax.gather(
            v, (i ^ d)[:, None], dnums, (1,),
            mode=lax.GatherScatterMode.PROMISE_IN_BOUNDS)
        v = v + perm
    return v


def _tree_sum(vs):
    vs = list(vs)
    while len(vs) > 1:
        vs = [vs[i] + vs[i + 1] for i in range(0, len(vs) - 1, 2)] + (
            [vs[-1]] if len(vs) % 2 else [])
    return vs[0]


def _rsqrt(d):
    i = lax.bitcast_convert_type(d, jnp.int32)
    i = jnp.int32(0x5F3759DF) - (i >> 1)
    y = lax.bitcast_convert_type(i, jnp.float32)
    return y * (1.5 - 0.5 * d * y * y)


def _sc_body(table_hbm, ids_hbm, pos_hbm, out_hbm, ids_v, pos_v,
             rows0, rows1, outv0, outv1, gsem0, gsem1, osem0, osem1):
    wid = lax.axis_index("s") * 2 + lax.axis_index("c")
    row0 = wid * _ROWS_PER_W
    rows = (rows0, rows1)
    outs = (outv0, outv1)
    gsems = (gsem0, gsem1)
    osems = (osem0, osem1)

    pltpu.sync_copy(ids_hbm.at[pl.ds(row0, _ROWS_PER_W)], ids_v)
    pltpu.sync_copy(pos_hbm.at[pl.ds(0, _SEQ * _HIDDEN)], pos_v)

    pltpu.async_copy(table_hbm.at[ids_v.at[pl.ds(0, _CHUNK)]], rows0, gsem0)

    def group_body(i, _):
        for b in (0, 1):
            c = 2 * i + b
            base = row0 + c * _CHUNK
            pltpu.make_async_copy(
                table_hbm.at[ids_v.at[pl.ds(c * _CHUNK, _CHUNK)]],
                rows[b], gsems[b]).wait()

            @pl.when(c + 1 < _NCHUNK)
            def _():
                idx = ids_v.at[pl.ds((c + 1) * _CHUNK, _CHUNK)]
                pltpu.async_copy(table_hbm.at[idx], rows[b ^ 1], gsems[b ^ 1])

            @pl.when(i >= 1)
            def _():
                pltpu.make_async_copy(
                    outs[b],
                    out_hbm.at[pl.ds(base - 2 * _CHUNK, _CHUNK)],
                    osems[b]).wait()

            rows_v = rows[b]
            out_v = outs[b]

            @plsc.parallel_loop(0, _CHUNK, unroll=4)
            def row_body(r):
                p = (base + r) % _SEQ
                pb = p * _HIDDEN
                xs = []
                acc = None
                sq = None
                for k in range(_HIDDEN // 16):
                    v = (rows_v[r, pl.ds(16 * k, 16)]
                         + pos_v[pl.ds(pb + 16 * k, 16)])
                    xs.append(v)
                    acc = v if acc is None else acc + v
                    sq = v * v if sq is None else sq + v * v
                tot = _lane_sum(acc)
                totsq = _lane_sum(sq)
                mean = tot * (1.0 / _HIDDEN)
                var = totsq * (1.0 / _HIDDEN) - mean * mean
                rinv = _rsqrt(var + 1e-3)
                mr = mean * rinv
                for k in range(_HIDDEN // 16):
                    out_v[r, pl.ds(16 * k, 16)] = xs[k] * rinv - mr

            pltpu.async_copy(out_v, out_hbm.at[pl.ds(base, _CHUNK)], osems[b])
        return 0

    lax.fori_loop(0, _NCHUNK // 2, group_body, 0)

    for b, cl in ((0, _NCHUNK - 2), (1, _NCHUNK - 1)):
        pltpu.make_async_copy(
            outs[b], out_hbm.at[pl.ds(row0 + cl * _CHUNK, _CHUNK)],
            osems[b]).wait()


@jax.jit
def _sc_call(word_embeddings, ids_flat, pos_flat):
    mesh = plsc.VectorSubcoreMesh(core_axis_name="c", subcore_axis_name="s")
    fn = functools.partial(
        pl.kernel,
        mesh=mesh,
        out_type=jax.ShapeDtypeStruct((_ROWS, _HIDDEN), jnp.float32),
        scratch_types=[
            pltpu.VMEM((_ROWS_PER_W,), jnp.int32),
            pltpu.VMEM((_SEQ * _HIDDEN,), jnp.float32),
            pltpu.VMEM((_CHUNK, _HIDDEN), jnp.float32),
            pltpu.VMEM((_CHUNK, _HIDDEN), jnp.float32),
            pltpu.VMEM((_CHUNK, _HIDDEN), jnp.float32),
            pltpu.VMEM((_CHUNK, _HIDDEN), jnp.float32),
            pltpu.SemaphoreType.DMA,
            pltpu.SemaphoreType.DMA,
            pltpu.SemaphoreType.DMA,
            pltpu.SemaphoreType.DMA,
        ],
    )(_sc_body)
    return fn(word_embeddings, ids_flat, pos_flat)


def kernel(input_ids, token_type_ids, word_embeddings, position_embeddings):
    del token_type_ids
    b, s = input_ids.shape
    ids_flat = input_ids.reshape(-1).astype(jnp.int32)
    pos_flat = position_embeddings.reshape(-1)
    out = _sc_call(word_embeddings, ids_flat, pos_flat)
    return out.reshape(b, s, _HIDDEN)


# speedup vs baseline: 2.6787x; 1.0468x over previous
import functools

import jax
import jax.numpy as jnp
from jax import lax
from jax.experimental import pallas as pl
from jax.experimental.pallas import tpu as pltpu
from jax.experimental.pallas import tpu_sc as plsc

_VOCAB = 100000
_HIDDEN = 128
_SEQ = 200
_BATCH = 1024
_ROWS = _BATCH * _SEQ
_NW = 32
_ROWS_PER_W = _ROWS // _NW
_CHUNK = 128
_NCHUNK = _ROWS_PER_W // _CHUNK


def _lane_sum(v):
    i = lax.iota(jnp.int32, 16)
    dnums = lax.GatherDimensionNumbers(
        offset_dims=(), collapsed_slice_dims=(0,), start_index_map=(0,))
    for d in (1, 2, 4, 8):
        perm = lax.gather(
            v, (i ^ d)[:, None], dnums, (1,),
            mode=lax.GatherScatterMode.PROMISE_IN_BOUNDS)
        v = v + perm
    return v


def _tree_sum(vs):
    vs = list(vs)
    while len(vs) > 1:
        vs = [vs[i] + vs[i + 1] for i in range(0, len(vs) - 1, 2)] + (
            [vs[-1]] if len(vs) % 2 else [])
    return vs[0]


def _rsqrt(d):
    i = lax.bitcast_convert_type(d, jnp.int32)
    i = jnp.int32(0x5F3759DF) - (i >> 1)
    y = lax.bitcast_convert_type(i, jnp.float32)
    return y * (1.5 - 0.5 * d * y * y)


def _sc_body(table_hbm, ids_hbm, pos_hbm, out_hbm, ids_v, pos_v,
             rows0, rows1, outv0, outv1, gsem0, gsem1, osem0, osem1):
    wid = lax.axis_index("s") * 2 + lax.axis_index("c")
    row0 = wid * _ROWS_PER_W
    rows = (rows0, rows1)
    outs = (outv0, outv1)
    gsems = (gsem0, gsem1)
    osems = (osem0, osem1)

    pltpu.sync_copy(ids_hbm.at[pl.ds(row0, _ROWS_PER_W)], ids_v)
    pltpu.sync_copy(pos_hbm.at[pl.ds(0, _SEQ * _HIDDEN)], pos_v)

    pltpu.async_copy(table_hbm.at[ids_v.at[pl.ds(0, _CHUNK)]], rows0, gsem0)

    def group_body(i, _):
        for b in (0, 1):
            c = 2 * i + b
            base = row0 + c * _CHUNK
            pltpu.make_async_copy(
                table_hbm.at[ids_v.at[pl.ds(c * _CHUNK, _CHUNK)]],
                rows[b], gsems[b]).wait()

            @pl.when(c + 1 < _NCHUNK)
            def _():
                idx = ids_v.at[pl.ds((c + 1) * _CHUNK, _CHUNK)]
                pltpu.async_copy(table_hbm.at[idx], rows[b ^ 1], gsems[b ^ 1])

            @pl.when(i >= 1)
            def _():
                pltpu.make_async_copy(
                    outs[b],
                    out_hbm.at[pl.ds(base - 2 * _CHUNK, _CHUNK)],
                    osems[b]).wait()

            rows_v = rows[b]
            out_v = outs[b]

            @plsc.parallel_loop(0, _CHUNK, unroll=4)
            def row_body(r):
                p = (base + r) % _SEQ
                pb = p * _HIDDEN
                xs = []
                acc = None
                sq = None
                for k in range(_HIDDEN // 16):
                    v = (rows_v[r, pl.ds(16 * k, 16)]
                         + pos_v[pl.ds(pb + 16 * k, 16)])
                    xs.append(v)
                    acc = v if acc is None else acc + v
                    sq = v * v if sq is None else sq + v * v
                tot = _lane_sum(acc)
                totsq = _lane_sum(sq)
                mean = tot * (1.0 / _HIDDEN)
                var = totsq * (1.0 / _HIDDEN) - mean * mean
                rinv = _rsqrt(var + 1e-3)
                for k in range(_HIDDEN // 16):
                    out_v[r, pl.ds(16 * k, 16)] = (xs[k] - mean) * rinv

            pltpu.async_copy(out_v, out_hbm.at[pl.ds(base, _CHUNK)], osems[b])
        return 0

    lax.fori_loop(0, _NCHUNK // 2, group_body, 0)

    for b, cl in ((0, _NCHUNK - 2), (1, _NCHUNK - 1)):
        pltpu.make_async_copy(
            outs[b], out_hbm.at[pl.ds(row0 + cl * _CHUNK, _CHUNK)],
            osems[b]).wait()


@jax.jit
def _sc_call(word_embeddings, ids_flat, pos_flat):
    mesh = plsc.VectorSubcoreMesh(core_axis_name="c", subcore_axis_name="s")
    fn = functools.partial(
        pl.kernel,
        mesh=mesh,
        out_type=jax.ShapeDtypeStruct((_ROWS, _HIDDEN), jnp.float32),
        scratch_types=[
            pltpu.VMEM((_ROWS_PER_W,), jnp.int32),
            pltpu.VMEM((_SEQ * _HIDDEN,), jnp.float32),
            pltpu.VMEM((_CHUNK, _HIDDEN), jnp.float32),
            pltpu.VMEM((_CHUNK, _HIDDEN), jnp.float32),
            pltpu.VMEM((_CHUNK, _HIDDEN), jnp.float32),
            pltpu.VMEM((_CHUNK, _HIDDEN), jnp.float32),
            pltpu.SemaphoreType.DMA,
            pltpu.SemaphoreType.DMA,
            pltpu.SemaphoreType.DMA,
            pltpu.SemaphoreType.DMA,
        ],
    )(_sc_body)
    return fn(word_embeddings, ids_flat, pos_flat)


def kernel(input_ids, token_type_ids, word_embeddings, position_embeddings):
    del token_type_ids
    b, s = input_ids.shape
    ids_flat = input_ids.reshape(-1).astype(jnp.int32)
    pos_flat = position_embeddings.reshape(-1)
    out = _sc_call(word_embeddings, ids_flat, pos_flat)
    return out.reshape(b, s, _HIDDEN)
